# Initial kernel scaffold; baseline (speedup 1.0000x reference)
import functools

import jax
import jax.numpy as jnp
from jax import lax
from jax.experimental import pallas as pl
from jax.experimental.pallas import tpu as pltpu
from jax.experimental.pallas import tpu_sc as plsc

N = 100000
C = 128
G = 64
NW = 32
B = 200
NBLK = N // B
MAXB = -(-NBLK // NW)
CCH = C // 16


def _sc_partials(feat_flat, ids):
    mesh = plsc.VectorSubcoreMesh(core_axis_name="c", subcore_axis_name="s")

    @functools.partial(
        pl.kernel,
        mesh=mesh,
        out_type=(
            jax.ShapeDtypeStruct((NW * G * C,), jnp.float32),
            jax.ShapeDtypeStruct((NW * G,), jnp.float32),
        ),
        scratch_types=[
            pltpu.VMEM((B * C,), jnp.float32),
            pltpu.VMEM((B,), jnp.int32),
            pltpu.VMEM((G * C,), jnp.float32),
            pltpu.VMEM((G,), jnp.float32),
        ],
    )
    def k(feat_hbm, ids_hbm, part_hbm, cnt_hbm, buf, idsb, acc, cnt):
        wid = lax.axis_index("s") * 2 + lax.axis_index("c")
        zero = jnp.zeros((16,), jnp.float32)
        iota = lax.iota(jnp.int32, 16)

        def zero_body(i, _):
            acc[pl.ds(i * 16, 16)] = zero
            return 0

        lax.fori_loop(0, G * C // 16, zero_body, 0)
        for q in range(G // 16):
            cnt[pl.ds(q * 16, 16)] = zero

        def blk_body(i, _):
            blk = wid + i * NW

            @pl.when(blk < NBLK)
            def _():
                start = blk * B
                pltpu.sync_copy(ids_hbm.at[pl.ds(start, B)], idsb)
                pltpu.sync_copy(feat_hbm.at[pl.ds(start * C, B * C)], buf)
                id0 = idsb[0]
                id1 = idsb[B - 1]

                @pl.when(id0 == id1)
                def _uniform():
                    def row(r, carry):
                        base = r * C
                        return tuple(
                            carry[c] + buf[pl.ds(base + c * 16, 16)]
                            for c in range(CCH)
                        )

                    sums = lax.fori_loop(
                        0, B, row, tuple(zero for _ in range(CCH))
                    )
                    abase = id0 * C
                    for c in range(CCH):
                        sl = pl.ds(abase + c * 16, 16)
                        acc[sl] = acc[sl] + sums[c]
                    cbase = (id0 // 16) * 16
                    csl = pl.ds(cbase, 16)
                    cnt[csl] = cnt[csl] + jnp.where(
                        iota + cbase == id0, float(B), 0.0
                    )

                @pl.when(id0 != id1)
                def _boundary():
                    def row(r, _):
                        idr = idsb[r]
                        abase = idr * C
                        rbase = r * C
                        for c in range(CCH):
                            sl = pl.ds(abase + c * 16, 16)
                            acc[sl] = acc[sl] + buf[pl.ds(rbase + c * 16, 16)]
                        cbase = (idr // 16) * 16
                        csl = pl.ds(cbase, 16)
                        cnt[csl] = cnt[csl] + jnp.where(
                            iota + cbase == idr, 1.0, 0.0
                        )
                        return 0

                    lax.fori_loop(0, B, row, 0)

            return 0

        lax.fori_loop(0, MAXB, blk_body, 0)
        pltpu.sync_copy(acc, part_hbm.at[pl.ds(wid * G * C, G * C)])
        pltpu.sync_copy(cnt, cnt_hbm.at[pl.ds(wid * G, G)])

    return k(feat_flat, ids)


def _combine(part, cnt):
    def body(part_ref, cnt_ref, out_ref):
        sums = jnp.sum(part_ref[...], axis=0)
        n = jnp.maximum(jnp.sum(cnt_ref[...], axis=0), 1.0)
        out_ref[...] = sums / n[:, None]

    return pl.pallas_call(
        body,
        out_shape=jax.ShapeDtypeStruct((G, C), jnp.float32),
    )(part, cnt)


def kernel(feat0, graph_ids):
    feat_flat = feat0.reshape(N * C)
    ids = graph_ids.astype(jnp.int32)
    part, cnt = _sc_partials(feat_flat, ids)
    return _combine(part.reshape(NW, G, C), cnt.reshape(NW, G))


# speedup vs baseline: 3.7190x; 3.7190x over previous
import functools

import jax
import jax.numpy as jnp
from jax import lax
from jax.experimental import pallas as pl
from jax.experimental.pallas import tpu as pltpu
from jax.experimental.pallas import tpu_sc as plsc

N = 100000
C = 128
G = 64
NW = 32
B = 160
NBLK = N // B
MAXB = -(-NBLK // NW)
CCH = C // 16


def _sc_partials(feat_flat, ids):
    mesh = plsc.VectorSubcoreMesh(core_axis_name="c", subcore_axis_name="s")

    @functools.partial(
        pl.kernel,
        mesh=mesh,
        out_type=(
            jax.ShapeDtypeStruct((NW * G * C,), jnp.float32),
            jax.ShapeDtypeStruct((NW * G,), jnp.float32),
        ),
        scratch_types=[
            pltpu.VMEM((B * C,), jnp.float32),
            pltpu.VMEM((B,), jnp.int32),
            pltpu.VMEM((G * C,), jnp.float32),
            pltpu.VMEM((G,), jnp.float32),
        ],
    )
    def k(feat_hbm, ids_hbm, part_hbm, cnt_hbm, buf, idsb, acc, cnt):
        wid = lax.axis_index("s") * 2 + lax.axis_index("c")
        zero = jnp.zeros((16,), jnp.float32)
        iota = lax.iota(jnp.int32, 16)

        def zero_body(i, _):
            acc[pl.ds(i * 16, 16)] = zero
            return 0

        lax.fori_loop(0, G * C // 16, zero_body, 0)
        for q in range(G // 16):
            cnt[pl.ds(q * 16, 16)] = zero

        def blk_body(i, _):
            blk = wid + i * NW

            @pl.when(blk < NBLK)
            def _():
                start = blk * B
                pltpu.sync_copy(ids_hbm.at[pl.ds(start, B)], idsb)
                pltpu.sync_copy(feat_hbm.at[pl.ds(start * C, B * C)], buf)
                id0 = idsb[pl.ds(0, 16)][0]
                id1 = idsb[pl.ds(B - 16, 16)][15]

                @pl.when(id0 == id1)
                def _uniform():
                    def row(r, carry):
                        base = r * C
                        return tuple(
                            carry[c] + buf[pl.ds(base + c * 16, 16)]
                            for c in range(CCH)
                        )

                    sums = lax.fori_loop(
                        0, B, row, tuple(zero for _ in range(CCH))
                    )
                    abase = id0 * C
                    for c in range(CCH):
                        sl = pl.ds(abase + c * 16, 16)
                        acc[sl] = acc[sl] + sums[c]
                    cbase = (id0 // 16) * 16
                    csl = pl.ds(cbase, 16)
                    cnt[csl] = cnt[csl] + jnp.where(
                        iota + cbase == id0, float(B), 0.0
                    )

                @pl.when(id0 != id1)
                def _boundary():
                    def grp(gi, _):
                        gbase = gi * 16
                        vids = idsb[pl.ds(gbase, 16)]
                        for j in range(16):
                            idr = vids[j]
                            abase = idr * C
                            rbase = (gbase + j) * C
                            for c in range(CCH):
                                sl = pl.ds(abase + c * 16, 16)
                                acc[sl] = (
                                    acc[sl] + buf[pl.ds(rbase + c * 16, 16)]
                                )
                            cbase = (idr // 16) * 16
                            csl = pl.ds(cbase, 16)
                            cnt[csl] = cnt[csl] + jnp.where(
                                iota + cbase == idr, 1.0, 0.0
                            )
                        return 0

                    lax.fori_loop(0, B // 16, grp, 0)

            return 0

        lax.fori_loop(0, MAXB, blk_body, 0)
        pltpu.sync_copy(acc, part_hbm.at[pl.ds(wid * G * C, G * C)])
        pltpu.sync_copy(cnt, cnt_hbm.at[pl.ds(wid * G, G)])

    return k(feat_flat, ids)


def _combine(part, cnt):
    def body(part_ref, cnt_ref, out_ref):
        sums = jnp.sum(part_ref[...], axis=0)
        n = jnp.maximum(jnp.sum(cnt_ref[...], axis=0), 1.0)
        out_ref[...] = sums / n[:, None]

    return pl.pallas_call(
        body,
        out_shape=jax.ShapeDtypeStruct((G, C), jnp.float32),
    )(part, cnt)


def kernel(feat0, graph_ids):
    feat_flat = feat0.reshape(N * C)
    ids = graph_ids.astype(jnp.int32)
    part, cnt = _sc_partials(feat_flat, ids)
    return _combine(part.reshape(NW, G, C), cnt.reshape(NW, G))


# speedup vs baseline: 4.9897x; 1.3417x over previous
import functools

import jax
import jax.numpy as jnp
from jax import lax
from jax.experimental import pallas as pl
from jax.experimental.pallas import tpu as pltpu
from jax.experimental.pallas import tpu_sc as plsc

N = 100000
C = 128
G = 64
NW = 32
B = 160
NBLK = N // B
MAXB = -(-NBLK // NW)
CCH = C // 16


def _sc_partials(feat_flat, ids):
    mesh = plsc.VectorSubcoreMesh(core_axis_name="c", subcore_axis_name="s")

    @functools.partial(
        pl.kernel,
        mesh=mesh,
        out_type=(
            jax.ShapeDtypeStruct((NW * G * C,), jnp.float32),
            jax.ShapeDtypeStruct((NW * G,), jnp.float32),
        ),
        scratch_types=[
            pltpu.VMEM((B * C,), jnp.float32),
            pltpu.VMEM((B * C,), jnp.float32),
            pltpu.VMEM((B,), jnp.int32),
            pltpu.VMEM((B,), jnp.int32),
            pltpu.VMEM((G * C,), jnp.float32),
            pltpu.VMEM((G,), jnp.float32),
            pltpu.SemaphoreType.DMA,
            pltpu.SemaphoreType.DMA,
        ],
    )
    def k(feat_hbm, ids_hbm, part_hbm, cnt_hbm,
          buf0, buf1, ids0, ids1, acc, cnt, sem0, sem1):
        wid = lax.axis_index("s") * 2 + lax.axis_index("c")
        zero = jnp.zeros((16,), jnp.float32)
        iota = lax.iota(jnp.int32, 16)

        def zero_body(i, _):
            acc[pl.ds(i * 16, 16)] = zero
            return 0

        lax.fori_loop(0, G * C // 16, zero_body, 0)
        for q in range(G // 16):
            cnt[pl.ds(q * 16, 16)] = zero

        def copies(blk, buf, idsb, sem):
            start = blk * B
            return (
                pltpu.make_async_copy(
                    ids_hbm.at[pl.ds(start, B)], idsb, sem
                ),
                pltpu.make_async_copy(
                    feat_hbm.at[pl.ds(start * C, B * C)], buf, sem
                ),
            )

        def fetch(blk, buf, idsb, sem):
            @pl.when(blk < NBLK)
            def _():
                for cp in copies(blk, buf, idsb, sem):
                    cp.start()

        def compute(blk, buf, idsb, sem):
            @pl.when(blk < NBLK)
            def _():
                for cp in copies(blk, buf, idsb, sem):
                    cp.wait()
                id0 = idsb[pl.ds(0, 16)][0]
                id1 = idsb[pl.ds(B - 16, 16)][15]

                @pl.when(id0 == id1)
                def _uniform():
                    def row(r, carry):
                        base = r * C
                        return tuple(
                            carry[c] + buf[pl.ds(base + c * 16, 16)]
                            for c in range(CCH)
                        )

                    sums = lax.fori_loop(
                        0, B, row, tuple(zero for _ in range(CCH))
                    )
                    abase = id0 * C
                    for c in range(CCH):
                        sl = pl.ds(abase + c * 16, 16)
                        acc[sl] = acc[sl] + sums[c]
                    cbase = (id0 // 16) * 16
                    csl = pl.ds(cbase, 16)
                    cnt[csl] = cnt[csl] + jnp.where(
                        iota + cbase == id0, float(B), 0.0
                    )

                @pl.when(id0 != id1)
                def _boundary():
                    def grp(gi, _):
                        gbase = gi * 16
                        vids = idsb[pl.ds(gbase, 16)]
                        for j in range(16):
                            idr = vids[j]
                            abase = idr * C
                            rbase = (gbase + j) * C
                            for c in range(CCH):
                                sl = pl.ds(abase + c * 16, 16)
                                acc[sl] = (
                                    acc[sl] + buf[pl.ds(rbase + c * 16, 16)]
                                )
                            cbase = (idr // 16) * 16
                            csl = pl.ds(cbase, 16)
                            cnt[csl] = cnt[csl] + jnp.where(
                                iota + cbase == idr, 1.0, 0.0
                            )
                        return 0

                    lax.fori_loop(0, B // 16, grp, 0)

        fetch(wid, buf0, ids0, sem0)

        def pair_body(p, _):
            i0 = 2 * p
            blk0 = wid + i0 * NW
            blk1 = wid + (i0 + 1) * NW
            blk2 = wid + (i0 + 2) * NW
            fetch(blk1, buf1, ids1, sem1)
            compute(blk0, buf0, ids0, sem0)
            fetch(blk2, buf0, ids0, sem0)
            compute(blk1, buf1, ids1, sem1)
            return 0

        lax.fori_loop(0, MAXB // 2, pair_body, 0)
        pltpu.sync_copy(acc, part_hbm.at[pl.ds(wid * G * C, G * C)])
        pltpu.sync_copy(cnt, cnt_hbm.at[pl.ds(wid * G, G)])

    return k(feat_flat, ids)


def _combine(part, cnt):
    def body(part_ref, cnt_ref, out_ref):
        sums = jnp.sum(part_ref[...], axis=0)
        n = jnp.maximum(jnp.sum(cnt_ref[...], axis=0), 1.0)
        out_ref[...] = sums / n[:, None]

    return pl.pallas_call(
        body,
        out_shape=jax.ShapeDtypeStruct((G, C), jnp.float32),
    )(part, cnt)


def kernel(feat0, graph_ids):
    feat_flat = feat0.reshape(N * C)
    ids = graph_ids.astype(jnp.int32)
    part, cnt = _sc_partials(feat_flat, ids)
    return _combine(part.reshape(NW, G, C), cnt.reshape(NW, G))
